# Initial kernel scaffold; baseline (speedup 1.0000x reference)
import jax
import jax.numpy as jnp
from jax import lax
from jax.experimental import pallas as pl
from jax.experimental.pallas import tpu as pltpu
from jax.experimental.pallas import tpu_sc as plsc

HD = 768
NE = 8
NTOK = 2048
FF = 2048
TBT = 128
NBT = NTOK // TBT
TB = 64
NB = NTOK * 2 // TB + NE
PAD = NB * TB
FC = 512
NFC = FF // FC
NC = 2
NS = 16
NW = NC * NS
TPW = NTOK // NW


def _router_body(x_ref, wg_ref, w0_ref, w1_ref, e0_ref, e1_ref, r0_ref,
                 r1_ref, off_ref, bexp_ref, aux_ref, cnt_ref, psum_ref):
    step = pl.program_id(0)

    @pl.when(step == 0)
    def _():
        cnt_ref[...] = jnp.zeros_like(cnt_ref)
        psum_ref[...] = jnp.zeros_like(psum_ref)

    xb = x_ref[...]
    logits = lax.dot_general(xb, wg_ref[...], (((1,), (1,)), ((), ())),
                             preferred_element_type=jnp.float32)
    col = lax.broadcasted_iota(jnp.int32, (TBT, 128), 1)
    row = lax.broadcasted_iota(jnp.int32, (TBT, 128), 0)
    valid = col < NE
    lg = jnp.where(valid, logits, jnp.float32(-1e30))
    m = jnp.max(lg, axis=1, keepdims=True)
    ex = jnp.where(valid, jnp.exp(lg - m), 0.0)
    probs = ex / jnp.sum(ex, axis=1, keepdims=True)

    p1 = jnp.max(jnp.where(valid, probs, -1.0), axis=1, keepdims=True)
    i1 = jnp.min(jnp.where(valid & (probs == p1), col, 128),
                 axis=1, keepdims=True)
    mask1 = col == i1
    pr2 = jnp.where(valid & ~mask1, probs, -1.0)
    p2 = jnp.max(pr2, axis=1, keepdims=True)
    i2 = jnp.min(jnp.where(pr2 == p2, col, 128), axis=1, keepdims=True)
    mask2 = col == i2

    denom = p1 + p2 + 1e-6
    w0_ref[...] = p1 / denom
    w1_ref[...] = p2 / denom
    e0_ref[...] = i1
    e1_ref[...] = i2

    mm = (mask1 | mask2).astype(jnp.float32)
    tri = (row > col).astype(jnp.float32)
    rank_blk = lax.dot_general(tri, mm, (((1,), (0,)), ((), ())),
                               preferred_element_type=jnp.float32)
    rank_full = rank_blk + cnt_ref[...]
    r0_ref[...] = jnp.sum(jnp.where(mask1, rank_full, 0.0), axis=1,
                          keepdims=True).astype(jnp.int32)
    r1_ref[...] = jnp.sum(jnp.where(mask2, rank_full, 0.0), axis=1,
                          keepdims=True).astype(jnp.int32)

    cnt_ref[...] = cnt_ref[...] + jnp.sum(mm, axis=0, keepdims=True)
    psum_ref[...] = psum_ref[...] + jnp.sum(probs, axis=0, keepdims=True)

    @pl.when(step == NBT - 1)
    def _():
        cnt = cnt_ref[...]
        sizes = jnp.floor((cnt + (TB - 1)) * (1.0 / TB)) * TB
        upper = (row < col).astype(jnp.float32)
        off = lax.dot_general(sizes, upper, (((1,), (0,)), ((), ())),
                              preferred_element_type=jnp.float32)
        off_ref[...] = off
        offmat = jnp.broadcast_to(off, (128, 128))
        pos = (row * TB).astype(jnp.float32)
        cmp = ((offmat <= pos) & (col < NE)).astype(jnp.float32)
        bexp_ref[...] = jnp.sum(cmp, axis=1, keepdims=True) - 1.0
        frac = cnt * (1.0 / NTOK)
        pmean = psum_ref[...] * (1.0 / NTOK)
        aux_ref[...] = jnp.full((1, 128), jnp.sum(frac * pmean) * NE,
                                jnp.float32)


def _run_router(x2, wg_pad, interpret=False):
    fn = pl.pallas_call(
        _router_body,
        grid=(NBT,),
        in_specs=[
            pl.BlockSpec((TBT, HD), lambda b: (b, 0)),
            pl.BlockSpec((128, HD), lambda b: (0, 0)),
        ],
        out_specs=[pl.BlockSpec((TBT, 1), lambda b: (b, 0))] * 6 + [
            pl.BlockSpec((1, 128), lambda b: (0, 0)),
            pl.BlockSpec((128, 1), lambda b: (0, 0)),
            pl.BlockSpec((1, 128), lambda b: (0, 0)),
        ],
        out_shape=[
            jax.ShapeDtypeStruct((NTOK, 1), jnp.float32),
            jax.ShapeDtypeStruct((NTOK, 1), jnp.float32),
            jax.ShapeDtypeStruct((NTOK, 1), jnp.int32),
            jax.ShapeDtypeStruct((NTOK, 1), jnp.int32),
            jax.ShapeDtypeStruct((NTOK, 1), jnp.int32),
            jax.ShapeDtypeStruct((NTOK, 1), jnp.int32),
            jax.ShapeDtypeStruct((1, 128), jnp.float32),
            jax.ShapeDtypeStruct((128, 1), jnp.float32),
            jax.ShapeDtypeStruct((1, 128), jnp.float32),
        ],
        scratch_shapes=[pltpu.VMEM((1, 128), jnp.float32),
                        pltpu.VMEM((1, 128), jnp.float32)],
        compiler_params=pltpu.CompilerParams(
            dimension_semantics=("arbitrary",)),
        interpret=interpret,
    )
    return fn(x2, wg_pad)


def _dispatch_body(x_hbm, e0_hbm, e1_hbm, r0_hbm, r1_hbm, off_hbm,
                   xs_hbm, d0_hbm, d1_hbm,
                   xbuf, e0v, e1v, r0v, r1v, offv, d0v, d1v, sem):
    wid = lax.axis_index("s") * NC + lax.axis_index("c")
    base = wid * TPW
    pltpu.sync_copy(x_hbm.at[pl.ds(base, TPW)], xbuf)
    pltpu.sync_copy(e0_hbm.at[pl.ds(base, TPW)], e0v)
    pltpu.sync_copy(e1_hbm.at[pl.ds(base, TPW)], e1v)
    pltpu.sync_copy(r0_hbm.at[pl.ds(base, TPW)], r0v)
    pltpu.sync_copy(r1_hbm.at[pl.ds(base, TPW)], r1v)
    pltpu.sync_copy(off_hbm, offv)
    for j in range(TPW // 16):
        sl = pl.ds(j * 16, 16)
        d0v[sl] = plsc.load_gather(offv, [e0v[sl]]) + r0v[sl]
        d1v[sl] = plsc.load_gather(offv, [e1v[sl]]) + r1v[sl]
    pltpu.async_copy(xbuf, xs_hbm.at[d0v], sem).wait()
    pltpu.async_copy(xbuf, xs_hbm.at[d1v], sem).wait()
    pltpu.sync_copy(d0v, d0_hbm.at[pl.ds(base, TPW)])
    pltpu.sync_copy(d1v, d1_hbm.at[pl.ds(base, TPW)])


def _run_dispatch(x2, e0, e1, r0, r1, off16):
    mesh = plsc.VectorSubcoreMesh(core_axis_name="c", subcore_axis_name="s",
                                  num_cores=NC, num_subcores=NS)
    fn = pl.kernel(
        _dispatch_body,
        out_type=[
            jax.ShapeDtypeStruct((PAD, HD), jnp.float32),
            jax.ShapeDtypeStruct((NTOK,), jnp.int32),
            jax.ShapeDtypeStruct((NTOK,), jnp.int32),
        ],
        mesh=mesh,
        scratch_types=[
            pltpu.VMEM((TPW, HD), jnp.float32),
            pltpu.VMEM((TPW,), jnp.int32),
            pltpu.VMEM((TPW,), jnp.int32),
            pltpu.VMEM((TPW,), jnp.int32),
            pltpu.VMEM((TPW,), jnp.int32),
            pltpu.VMEM((16,), jnp.int32),
            pltpu.VMEM((TPW,), jnp.int32),
            pltpu.VMEM((TPW,), jnp.int32),
            pltpu.SemaphoreType.DMA,
        ],
    )
    return fn(x2, e0, e1, r0, r1, off16)


def _ffn_body(bexp_ref, xs_ref, w1_ref, w3_ref, w2_ref, y_ref):
    f = pl.program_id(1)
    xb = xs_ref[...]
    h1 = lax.dot_general(xb, w1_ref[0], (((1,), (1,)), ((), ())),
                         preferred_element_type=jnp.float32)
    h3 = lax.dot_general(xb, w3_ref[0], (((1,), (1,)), ((), ())),
                         preferred_element_type=jnp.float32)
    inner = h1 * jax.nn.sigmoid(h1) * h3
    part = lax.dot_general(inner, w2_ref[0], (((1,), (1,)), ((), ())),
                           preferred_element_type=jnp.float32)

    @pl.when(f == 0)
    def _():
        y_ref[...] = part

    @pl.when(f != 0)
    def _():
        y_ref[...] = y_ref[...] + part


def _run_ffn(bexp_i, xs, W1, W3, W2, interpret=False):
    grid_spec = pltpu.PrefetchScalarGridSpec(
        num_scalar_prefetch=1,
        grid=(NB, NFC),
        in_specs=[
            pl.BlockSpec((TB, HD), lambda b, f, bexp: (b, 0)),
            pl.BlockSpec((1, FC, HD), lambda b, f, bexp: (bexp[b], f, 0)),
            pl.BlockSpec((1, FC, HD), lambda b, f, bexp: (bexp[b], f, 0)),
            pl.BlockSpec((1, HD, FC), lambda b, f, bexp: (bexp[b], 0, f)),
        ],
        out_specs=pl.BlockSpec((TB, HD), lambda b, f, bexp: (b, 0)),
    )
    fn = pl.pallas_call(
        _ffn_body,
        grid_spec=grid_spec,
        out_shape=jax.ShapeDtypeStruct((PAD, HD), jnp.float32),
        compiler_params=pltpu.CompilerParams(
            dimension_semantics=("arbitrary", "arbitrary")),
        interpret=interpret,
    )
    return fn(bexp_i, xs, W1, W3, W2)


def _combine_body(y_hbm, d0_hbm, d1_hbm, w0_hbm, w1_hbm, out_hbm,
                  d0v, d1v, w0v, w1v, buf0, buf1, sem):
    wid = lax.axis_index("s") * NC + lax.axis_index("c")
    base = wid * TPW
    pltpu.sync_copy(d0_hbm.at[pl.ds(base, TPW)], d0v)
    pltpu.sync_copy(d1_hbm.at[pl.ds(base, TPW)], d1v)
    pltpu.sync_copy(w0_hbm.at[pl.ds(base, TPW)], w0v)
    pltpu.sync_copy(w1_hbm.at[pl.ds(base, TPW)], w1v)
    pltpu.async_copy(y_hbm.at[d0v], buf0, sem).wait()
    pltpu.async_copy(y_hbm.at[d1v], buf1, sem).wait()

    def tok(t, carry):
        ti = jnp.full((16,), t, jnp.int32)
        w0b = plsc.load_gather(w0v, [ti])
        w1b = plsc.load_gather(w1v, [ti])
        for j in range(HD // 16):
            sl = pl.ds(j * 16, 16)
            buf0[t, sl] = buf0[t, sl] * w0b + buf1[t, sl] * w1b
        return carry

    lax.fori_loop(0, TPW, tok, 0)
    pltpu.sync_copy(buf0, out_hbm.at[pl.ds(base, TPW)])


def _run_combine(y, d0, d1, w0, w1):
    mesh = plsc.VectorSubcoreMesh(core_axis_name="c", subcore_axis_name="s",
                                  num_cores=NC, num_subcores=NS)
    fn = pl.kernel(
        _combine_body,
        out_type=jax.ShapeDtypeStruct((NTOK, HD), jnp.float32),
        mesh=mesh,
        scratch_types=[
            pltpu.VMEM((TPW,), jnp.int32),
            pltpu.VMEM((TPW,), jnp.int32),
            pltpu.VMEM((TPW,), jnp.float32),
            pltpu.VMEM((TPW,), jnp.float32),
            pltpu.VMEM((TPW, HD), jnp.float32),
            pltpu.VMEM((TPW, HD), jnp.float32),
            pltpu.SemaphoreType.DMA,
        ],
    )
    return fn(y, d0, d1, w0, w1)


def kernel(x, Wg, W1, W2, W3):
    x2 = x.reshape(NTOK, HD)
    wg_pad = jnp.zeros((128, HD), jnp.float32).at[:NE].set(Wg)
    w0, w1, e0, e1, r0, r1, off, bexp, aux = _run_router(x2, wg_pad)
    off16 = off[0, :16].astype(jnp.int32)
    bexp_i = bexp[:NB, 0].astype(jnp.int32)
    xs, d0, d1 = _run_dispatch(x2, e0.reshape(NTOK), e1.reshape(NTOK),
                               r0.reshape(NTOK), r1.reshape(NTOK), off16)
    y = _run_ffn(bexp_i, xs, W1, W3, W2)
    out2 = _run_combine(y, d0, d1, w0.reshape(NTOK), w1.reshape(NTOK))
    return out2.reshape(1, NTOK, HD), aux[0, 0]


# speedup vs baseline: 1.0952x; 1.0952x over previous
import jax
import jax.numpy as jnp
from jax import lax
from jax.experimental import pallas as pl
from jax.experimental.pallas import tpu as pltpu
from jax.experimental.pallas import tpu_sc as plsc

HD = 768
NE = 8
NTOK = 2048
FF = 2048
TBT = 128
NBT = NTOK // TBT
TB = 64
NB = NTOK * 2 // TB + NE
PAD = NB * TB
FC = 512
NFC = FF // FC
NC = 2
NS = 16
NW = NC * NS
TPW = NTOK // NW


def _router_body(x_ref, wg_ref, w0_ref, w1_ref, e0_ref, e1_ref, r0_ref,
                 r1_ref, off_ref, bexp_ref, aux_ref, cnt_ref, psum_ref):
    step = pl.program_id(0)

    @pl.when(step == 0)
    def _():
        cnt_ref[...] = jnp.zeros_like(cnt_ref)
        psum_ref[...] = jnp.zeros_like(psum_ref)

    xb = x_ref[...]
    logits = lax.dot_general(xb, wg_ref[...], (((1,), (1,)), ((), ())),
                             preferred_element_type=jnp.float32)
    col = lax.broadcasted_iota(jnp.int32, (TBT, 128), 1)
    row = lax.broadcasted_iota(jnp.int32, (TBT, 128), 0)
    valid = col < NE
    lg = jnp.where(valid, logits, jnp.float32(-1e30))
    m = jnp.max(lg, axis=1, keepdims=True)
    ex = jnp.where(valid, jnp.exp(lg - m), 0.0)
    probs = ex / jnp.sum(ex, axis=1, keepdims=True)

    p1 = jnp.max(jnp.where(valid, probs, -1.0), axis=1, keepdims=True)
    i1 = jnp.min(jnp.where(valid & (probs == p1), col, 128),
                 axis=1, keepdims=True)
    mask1 = col == i1
    pr2 = jnp.where(valid & ~mask1, probs, -1.0)
    p2 = jnp.max(pr2, axis=1, keepdims=True)
    i2 = jnp.min(jnp.where(pr2 == p2, col, 128), axis=1, keepdims=True)
    mask2 = col == i2

    denom = p1 + p2 + 1e-6
    w0_ref[...] = p1 / denom
    w1_ref[...] = p2 / denom
    e0_ref[...] = i1
    e1_ref[...] = i2

    mm = (mask1 | mask2).astype(jnp.float32)
    tri = (row > col).astype(jnp.float32)
    rank_blk = lax.dot_general(tri, mm, (((1,), (0,)), ((), ())),
                               preferred_element_type=jnp.float32)
    rank_full = rank_blk + cnt_ref[...]
    r0_ref[...] = jnp.sum(jnp.where(mask1, rank_full, 0.0), axis=1,
                          keepdims=True).astype(jnp.int32)
    r1_ref[...] = jnp.sum(jnp.where(mask2, rank_full, 0.0), axis=1,
                          keepdims=True).astype(jnp.int32)

    cnt_ref[...] = cnt_ref[...] + jnp.sum(mm, axis=0, keepdims=True)
    psum_ref[...] = psum_ref[...] + jnp.sum(probs, axis=0, keepdims=True)

    @pl.when(step == NBT - 1)
    def _():
        cnt = cnt_ref[...]
        sizes = jnp.floor((cnt + (TB - 1)) * (1.0 / TB)) * TB
        upper = (row < col).astype(jnp.float32)
        off = lax.dot_general(sizes, upper, (((1,), (0,)), ((), ())),
                              preferred_element_type=jnp.float32)
        off_ref[...] = off
        offmat = jnp.broadcast_to(off, (128, 128))
        pos = (row * TB).astype(jnp.float32)
        cmp = ((offmat <= pos) & (col < NE)).astype(jnp.float32)
        bexp_ref[...] = jnp.sum(cmp, axis=1, keepdims=True) - 1.0
        frac = cnt * (1.0 / NTOK)
        pmean = psum_ref[...] * (1.0 / NTOK)
        aux_ref[...] = jnp.full((1, 128), jnp.sum(frac * pmean) * NE,
                                jnp.float32)


def _run_router(x2, wg_pad, interpret=False):
    fn = pl.pallas_call(
        _router_body,
        grid=(NBT,),
        in_specs=[
            pl.BlockSpec((TBT, HD), lambda b: (b, 0)),
            pl.BlockSpec((128, HD), lambda b: (0, 0)),
        ],
        out_specs=[pl.BlockSpec((TBT, 1), lambda b: (b, 0))] * 6 + [
            pl.BlockSpec((1, 128), lambda b: (0, 0)),
            pl.BlockSpec((128, 1), lambda b: (0, 0)),
            pl.BlockSpec((1, 128), lambda b: (0, 0)),
        ],
        out_shape=[
            jax.ShapeDtypeStruct((NTOK, 1), jnp.float32),
            jax.ShapeDtypeStruct((NTOK, 1), jnp.float32),
            jax.ShapeDtypeStruct((NTOK, 1), jnp.int32),
            jax.ShapeDtypeStruct((NTOK, 1), jnp.int32),
            jax.ShapeDtypeStruct((NTOK, 1), jnp.int32),
            jax.ShapeDtypeStruct((NTOK, 1), jnp.int32),
            jax.ShapeDtypeStruct((1, 128), jnp.float32),
            jax.ShapeDtypeStruct((128, 1), jnp.float32),
            jax.ShapeDtypeStruct((1, 128), jnp.float32),
        ],
        scratch_shapes=[pltpu.VMEM((1, 128), jnp.float32),
                        pltpu.VMEM((1, 128), jnp.float32)],
        compiler_params=pltpu.CompilerParams(
            dimension_semantics=("arbitrary",)),
        interpret=interpret,
    )
    return fn(x2, wg_pad)


def _dispatch_body(x_hbm, e0_hbm, e1_hbm, r0_hbm, r1_hbm, off_hbm,
                   xs_hbm, d0_hbm, d1_hbm,
                   xbuf, e0v, e1v, r0v, r1v, offv, d0v, d1v, sem):
    wid = lax.axis_index("s") * NC + lax.axis_index("c")
    base = wid * TPW
    pltpu.sync_copy(x_hbm.at[pl.ds(base, TPW)], xbuf)
    pltpu.sync_copy(e0_hbm.at[pl.ds(base, TPW)], e0v)
    pltpu.sync_copy(e1_hbm.at[pl.ds(base, TPW)], e1v)
    pltpu.sync_copy(r0_hbm.at[pl.ds(base, TPW)], r0v)
    pltpu.sync_copy(r1_hbm.at[pl.ds(base, TPW)], r1v)
    pltpu.sync_copy(off_hbm, offv)
    for j in range(TPW // 16):
        sl = pl.ds(j * 16, 16)
        d0v[sl] = plsc.load_gather(offv, [e0v[sl]]) + r0v[sl]
        d1v[sl] = plsc.load_gather(offv, [e1v[sl]]) + r1v[sl]
    pltpu.async_copy(xbuf, xs_hbm.at[d0v], sem).wait()
    pltpu.async_copy(xbuf, xs_hbm.at[d1v], sem).wait()
    pltpu.sync_copy(d0v, d0_hbm.at[pl.ds(base, TPW)])
    pltpu.sync_copy(d1v, d1_hbm.at[pl.ds(base, TPW)])


def _run_dispatch(x2, e0, e1, r0, r1, off16):
    mesh = plsc.VectorSubcoreMesh(core_axis_name="c", subcore_axis_name="s",
                                  num_cores=NC, num_subcores=NS)
    fn = pl.kernel(
        _dispatch_body,
        out_type=[
            jax.ShapeDtypeStruct((PAD, HD), jnp.float32),
            jax.ShapeDtypeStruct((NTOK,), jnp.int32),
            jax.ShapeDtypeStruct((NTOK,), jnp.int32),
        ],
        mesh=mesh,
        scratch_types=[
            pltpu.VMEM((TPW, HD), jnp.float32),
            pltpu.VMEM((TPW,), jnp.int32),
            pltpu.VMEM((TPW,), jnp.int32),
            pltpu.VMEM((TPW,), jnp.int32),
            pltpu.VMEM((TPW,), jnp.int32),
            pltpu.VMEM((16,), jnp.int32),
            pltpu.VMEM((TPW,), jnp.int32),
            pltpu.VMEM((TPW,), jnp.int32),
            pltpu.SemaphoreType.DMA,
        ],
        compiler_params=pltpu.CompilerParams(needs_layout_passes=False),
    )
    return fn(x2, e0, e1, r0, r1, off16)


def _ffn_body(bexp_ref, xs_ref, w1_ref, w3_ref, w2_ref, y_ref):
    f = pl.program_id(1)
    xb = xs_ref[...]
    h1 = lax.dot_general(xb, w1_ref[0], (((1,), (1,)), ((), ())),
                         preferred_element_type=jnp.float32)
    h3 = lax.dot_general(xb, w3_ref[0], (((1,), (1,)), ((), ())),
                         preferred_element_type=jnp.float32)
    inner = h1 * jax.nn.sigmoid(h1) * h3
    part = lax.dot_general(inner, w2_ref[0], (((1,), (1,)), ((), ())),
                           preferred_element_type=jnp.float32)

    @pl.when(f == 0)
    def _():
        y_ref[...] = part

    @pl.when(f != 0)
    def _():
        y_ref[...] = y_ref[...] + part


def _run_ffn(bexp_i, xs, W1, W3, W2, interpret=False):
    grid_spec = pltpu.PrefetchScalarGridSpec(
        num_scalar_prefetch=1,
        grid=(NB, NFC),
        in_specs=[
            pl.BlockSpec((TB, HD), lambda b, f, bexp: (b, 0)),
            pl.BlockSpec((1, FC, HD), lambda b, f, bexp: (bexp[b], f, 0)),
            pl.BlockSpec((1, FC, HD), lambda b, f, bexp: (bexp[b], f, 0)),
            pl.BlockSpec((1, HD, FC), lambda b, f, bexp: (bexp[b], 0, f)),
        ],
        out_specs=pl.BlockSpec((TB, HD), lambda b, f, bexp: (b, 0)),
    )
    fn = pl.pallas_call(
        _ffn_body,
        grid_spec=grid_spec,
        out_shape=jax.ShapeDtypeStruct((PAD, HD), jnp.float32),
        compiler_params=pltpu.CompilerParams(
            dimension_semantics=("arbitrary", "arbitrary")),
        interpret=interpret,
    )
    return fn(bexp_i, xs, W1, W3, W2)


def _combine_body(y_hbm, d0_hbm, d1_hbm, w0_hbm, w1_hbm, out_hbm,
                  d0v, d1v, w0v, w1v, buf0, buf1, sem):
    wid = lax.axis_index("s") * NC + lax.axis_index("c")
    base = wid * TPW
    pltpu.sync_copy(d0_hbm.at[pl.ds(base, TPW)], d0v)
    pltpu.sync_copy(d1_hbm.at[pl.ds(base, TPW)], d1v)
    pltpu.sync_copy(w0_hbm.at[pl.ds(base, TPW)], w0v)
    pltpu.sync_copy(w1_hbm.at[pl.ds(base, TPW)], w1v)
    pltpu.async_copy(y_hbm.at[d0v], buf0, sem).wait()
    pltpu.async_copy(y_hbm.at[d1v], buf1, sem).wait()

    def tok(t, carry):
        ti = jnp.full((16,), t, jnp.int32)
        w0b = plsc.load_gather(w0v, [ti])
        w1b = plsc.load_gather(w1v, [ti])
        for j in range(HD // 16):
            sl = pl.ds(j * 16, 16)
            buf0[t, sl] = buf0[t, sl] * w0b + buf1[t, sl] * w1b
        return carry

    lax.fori_loop(0, TPW, tok, 0)
    pltpu.sync_copy(buf0, out_hbm.at[pl.ds(base, TPW)])


def _run_combine(y, d0, d1, w0, w1):
    mesh = plsc.VectorSubcoreMesh(core_axis_name="c", subcore_axis_name="s",
                                  num_cores=NC, num_subcores=NS)
    fn = pl.kernel(
        _combine_body,
        out_type=jax.ShapeDtypeStruct((NTOK, HD), jnp.float32),
        mesh=mesh,
        scratch_types=[
            pltpu.VMEM((TPW,), jnp.int32),
            pltpu.VMEM((TPW,), jnp.int32),
            pltpu.VMEM((TPW,), jnp.float32),
            pltpu.VMEM((TPW,), jnp.float32),
            pltpu.VMEM((TPW, HD), jnp.float32),
            pltpu.VMEM((TPW, HD), jnp.float32),
            pltpu.SemaphoreType.DMA,
        ],
        compiler_params=pltpu.CompilerParams(needs_layout_passes=False),
    )
    return fn(y, d0, d1, w0, w1)


def kernel(x, Wg, W1, W2, W3):
    x2 = x.reshape(NTOK, HD)
    wg_pad = jnp.zeros((128, HD), jnp.float32).at[:NE].set(Wg)
    w0, w1, e0, e1, r0, r1, off, bexp, aux = _run_router(x2, wg_pad)
    off16 = off[0, :16].astype(jnp.int32)
    bexp_i = bexp[:NB, 0].astype(jnp.int32)
    xs, d0, d1 = _run_dispatch(x2, e0.reshape(NTOK), e1.reshape(NTOK),
                               r0.reshape(NTOK), r1.reshape(NTOK), off16)
    y = _run_ffn(bexp_i, xs, W1, W3, W2)
    out2 = _run_combine(y, d0, d1, w0.reshape(NTOK), w1.reshape(NTOK))
    return out2.reshape(1, NTOK, HD), aux[0, 0]


# speedup vs baseline: 1.5448x; 1.4104x over previous
import jax
import jax.numpy as jnp
from jax import lax
from jax.experimental import pallas as pl
from jax.experimental.pallas import tpu as pltpu
from jax.experimental.pallas import tpu_sc as plsc

HD = 768
NE = 8
NTOK = 2048
FF = 2048
TBT = 128
NBT = NTOK // TBT
TB = 128
NB = NTOK * 2 // TB + NE
PAD = NB * TB
NC = 2
NS = 16
NW = NC * NS
TPW = NTOK // NW


def _router_body(x_ref, wg_ref, w0_ref, w1_ref, e0_ref, e1_ref, r0_ref,
                 r1_ref, off_ref, bexp_ref, aux_ref, cnt_ref, psum_ref):
    step = pl.program_id(0)

    @pl.when(step == 0)
    def _():
        cnt_ref[...] = jnp.zeros_like(cnt_ref)
        psum_ref[...] = jnp.zeros_like(psum_ref)

    xb = x_ref[...]
    logits = lax.dot_general(xb, wg_ref[...], (((1,), (1,)), ((), ())),
                             preferred_element_type=jnp.float32)
    col = lax.broadcasted_iota(jnp.int32, (TBT, 128), 1)
    row = lax.broadcasted_iota(jnp.int32, (TBT, 128), 0)
    valid = col < NE
    lg = jnp.where(valid, logits, jnp.float32(-1e30))
    m = jnp.max(lg, axis=1, keepdims=True)
    ex = jnp.where(valid, jnp.exp(lg - m), 0.0)
    probs = ex / jnp.sum(ex, axis=1, keepdims=True)

    p1 = jnp.max(jnp.where(valid, probs, -1.0), axis=1, keepdims=True)
    i1 = jnp.min(jnp.where(valid & (probs == p1), col, 128),
                 axis=1, keepdims=True)
    mask1 = col == i1
    pr2 = jnp.where(valid & ~mask1, probs, -1.0)
    p2 = jnp.max(pr2, axis=1, keepdims=True)
    i2 = jnp.min(jnp.where(pr2 == p2, col, 128), axis=1, keepdims=True)
    mask2 = col == i2

    denom = p1 + p2 + 1e-6
    w0_ref[...] = p1 / denom
    w1_ref[...] = p2 / denom
    e0_ref[...] = i1
    e1_ref[...] = i2

    mm = (mask1 | mask2).astype(jnp.float32)
    tri = (row > col).astype(jnp.float32)
    rank_blk = lax.dot_general(tri, mm, (((1,), (0,)), ((), ())),
                               preferred_element_type=jnp.float32)
    rank_full = rank_blk + cnt_ref[...]
    r0_ref[...] = jnp.sum(jnp.where(mask1, rank_full, 0.0), axis=1,
                          keepdims=True).astype(jnp.int32)
    r1_ref[...] = jnp.sum(jnp.where(mask2, rank_full, 0.0), axis=1,
                          keepdims=True).astype(jnp.int32)

    cnt_ref[...] = cnt_ref[...] + jnp.sum(mm, axis=0, keepdims=True)
    psum_ref[...] = psum_ref[...] + jnp.sum(probs, axis=0, keepdims=True)

    @pl.when(step == NBT - 1)
    def _():
        cnt = cnt_ref[...]
        sizes = jnp.floor((cnt + (TB - 1)) * (1.0 / TB)) * TB
        upper = (row < col).astype(jnp.float32)
        off = lax.dot_general(sizes, upper, (((1,), (0,)), ((), ())),
                              preferred_element_type=jnp.float32)
        off_ref[...] = off
        offmat = jnp.broadcast_to(off, (128, 128))
        pos = (row * TB).astype(jnp.float32)
        cmp = ((offmat <= pos) & (col < NE)).astype(jnp.float32)
        bexp_ref[...] = jnp.sum(cmp, axis=1, keepdims=True) - 1.0
        frac = cnt * (1.0 / NTOK)
        pmean = psum_ref[...] * (1.0 / NTOK)
        aux_ref[...] = jnp.full((1, 128), jnp.sum(frac * pmean) * NE,
                                jnp.float32)


def _run_router(x2, wg_pad, interpret=False):
    fn = pl.pallas_call(
        _router_body,
        grid=(NBT,),
        in_specs=[
            pl.BlockSpec((TBT, HD), lambda b: (b, 0)),
            pl.BlockSpec((128, HD), lambda b: (0, 0)),
        ],
        out_specs=[pl.BlockSpec((TBT, 1), lambda b: (b, 0))] * 6 + [
            pl.BlockSpec((1, 128), lambda b: (0, 0)),
            pl.BlockSpec((128, 1), lambda b: (0, 0)),
            pl.BlockSpec((1, 128), lambda b: (0, 0)),
        ],
        out_shape=[
            jax.ShapeDtypeStruct((NTOK, 1), jnp.float32),
            jax.ShapeDtypeStruct((NTOK, 1), jnp.float32),
            jax.ShapeDtypeStruct((NTOK, 1), jnp.int32),
            jax.ShapeDtypeStruct((NTOK, 1), jnp.int32),
            jax.ShapeDtypeStruct((NTOK, 1), jnp.int32),
            jax.ShapeDtypeStruct((NTOK, 1), jnp.int32),
            jax.ShapeDtypeStruct((1, 128), jnp.float32),
            jax.ShapeDtypeStruct((128, 1), jnp.float32),
            jax.ShapeDtypeStruct((1, 128), jnp.float32),
        ],
        scratch_shapes=[pltpu.VMEM((1, 128), jnp.float32),
                        pltpu.VMEM((1, 128), jnp.float32)],
        compiler_params=pltpu.CompilerParams(
            dimension_semantics=("arbitrary",)),
        interpret=interpret,
    )
    return fn(x2, wg_pad)


def _dispatch_body(x_hbm, e0_hbm, e1_hbm, r0_hbm, r1_hbm, off_hbm,
                   xs_hbm, d0_hbm, d1_hbm,
                   xbuf, e0v, e1v, r0v, r1v, offv, d0v, d1v, sem):
    wid = lax.axis_index("s") * NC + lax.axis_index("c")
    base = wid * TPW
    pltpu.sync_copy(x_hbm.at[pl.ds(base, TPW)], xbuf)
    pltpu.sync_copy(e0_hbm.at[pl.ds(base, TPW)], e0v)
    pltpu.sync_copy(e1_hbm.at[pl.ds(base, TPW)], e1v)
    pltpu.sync_copy(r0_hbm.at[pl.ds(base, TPW)], r0v)
    pltpu.sync_copy(r1_hbm.at[pl.ds(base, TPW)], r1v)
    pltpu.sync_copy(off_hbm, offv)
    for j in range(TPW // 16):
        sl = pl.ds(j * 16, 16)
        d0v[sl] = plsc.load_gather(offv, [e0v[sl]]) + r0v[sl]
        d1v[sl] = plsc.load_gather(offv, [e1v[sl]]) + r1v[sl]
    pltpu.async_copy(xbuf, xs_hbm.at[d0v], sem).wait()
    pltpu.async_copy(xbuf, xs_hbm.at[d1v], sem).wait()
    pltpu.sync_copy(d0v, d0_hbm.at[pl.ds(base, TPW)])
    pltpu.sync_copy(d1v, d1_hbm.at[pl.ds(base, TPW)])


def _run_dispatch(x2, e0, e1, r0, r1, off16):
    mesh = plsc.VectorSubcoreMesh(core_axis_name="c", subcore_axis_name="s",
                                  num_cores=NC, num_subcores=NS)
    fn = pl.kernel(
        _dispatch_body,
        out_type=[
            jax.ShapeDtypeStruct((PAD, HD // 2), jnp.int32),
            jax.ShapeDtypeStruct((NTOK,), jnp.int32),
            jax.ShapeDtypeStruct((NTOK,), jnp.int32),
        ],
        mesh=mesh,
        scratch_types=[
            pltpu.VMEM((TPW, HD // 2), jnp.int32),
            pltpu.VMEM((TPW,), jnp.int32),
            pltpu.VMEM((TPW,), jnp.int32),
            pltpu.VMEM((TPW,), jnp.int32),
            pltpu.VMEM((TPW,), jnp.int32),
            pltpu.VMEM((16,), jnp.int32),
            pltpu.VMEM((TPW,), jnp.int32),
            pltpu.VMEM((TPW,), jnp.int32),
            pltpu.SemaphoreType.DMA,
        ],
        compiler_params=pltpu.CompilerParams(needs_layout_passes=False),
    )
    return fn(x2, e0, e1, r0, r1, off16)


def _ffn_body(bexp_ref, xs_ref, w1_ref, w3_ref, w2_ref, y_ref):
    xb = xs_ref[...]
    h1 = lax.dot_general(xb, w1_ref[0], (((1,), (1,)), ((), ())),
                         preferred_element_type=jnp.float32)
    h3 = lax.dot_general(xb, w3_ref[0], (((1,), (1,)), ((), ())),
                         preferred_element_type=jnp.float32)
    inner = (h1 * jax.nn.sigmoid(h1) * h3).astype(jnp.bfloat16)
    y_ref[...] = lax.dot_general(inner, w2_ref[0], (((1,), (1,)), ((), ())),
                                 preferred_element_type=jnp.float32)


def _run_ffn(bexp_i, xs, W1, W3, W2, interpret=False):
    grid_spec = pltpu.PrefetchScalarGridSpec(
        num_scalar_prefetch=1,
        grid=(NB,),
        in_specs=[
            pl.BlockSpec((TB, HD), lambda b, bexp: (b, 0)),
            pl.BlockSpec((1, FF, HD), lambda b, bexp: (bexp[b], 0, 0)),
            pl.BlockSpec((1, FF, HD), lambda b, bexp: (bexp[b], 0, 0)),
            pl.BlockSpec((1, HD, FF), lambda b, bexp: (bexp[b], 0, 0)),
        ],
        out_specs=pl.BlockSpec((TB, HD), lambda b, bexp: (b, 0)),
    )
    fn = pl.pallas_call(
        _ffn_body,
        grid_spec=grid_spec,
        out_shape=jax.ShapeDtypeStruct((PAD, HD), jnp.float32),
        compiler_params=pltpu.CompilerParams(
            dimension_semantics=("arbitrary",)),
        interpret=interpret,
    )
    return fn(bexp_i, xs, W1, W3, W2)


def _combine_body(y_hbm, d0_hbm, d1_hbm, w0_hbm, w1_hbm, out_hbm,
                  d0v, d1v, w0v, w1v, buf0, buf1, sem):
    wid = lax.axis_index("s") * NC + lax.axis_index("c")
    base = wid * TPW
    pltpu.sync_copy(d0_hbm.at[pl.ds(base, TPW)], d0v)
    pltpu.sync_copy(d1_hbm.at[pl.ds(base, TPW)], d1v)
    pltpu.sync_copy(w0_hbm.at[pl.ds(base, TPW)], w0v)
    pltpu.sync_copy(w1_hbm.at[pl.ds(base, TPW)], w1v)
    pltpu.async_copy(y_hbm.at[d0v], buf0, sem).wait()
    pltpu.async_copy(y_hbm.at[d1v], buf1, sem).wait()

    def tok(t, carry):
        ti = jnp.full((16,), t, jnp.int32)
        w0b = plsc.load_gather(w0v, [ti])
        w1b = plsc.load_gather(w1v, [ti])
        for j in range(HD // 16):
            sl = pl.ds(j * 16, 16)
            buf0[t, sl] = buf0[t, sl] * w0b + buf1[t, sl] * w1b
        return carry

    lax.fori_loop(0, TPW, tok, 0)
    pltpu.sync_copy(buf0, out_hbm.at[pl.ds(base, TPW)])


def _run_combine(y, d0, d1, w0, w1):
    mesh = plsc.VectorSubcoreMesh(core_axis_name="c", subcore_axis_name="s",
                                  num_cores=NC, num_subcores=NS)
    fn = pl.kernel(
        _combine_body,
        out_type=jax.ShapeDtypeStruct((NTOK, HD), jnp.float32),
        mesh=mesh,
        scratch_types=[
            pltpu.VMEM((TPW,), jnp.int32),
            pltpu.VMEM((TPW,), jnp.int32),
            pltpu.VMEM((TPW,), jnp.float32),
            pltpu.VMEM((TPW,), jnp.float32),
            pltpu.VMEM((TPW, HD), jnp.float32),
            pltpu.VMEM((TPW, HD), jnp.float32),
            pltpu.SemaphoreType.DMA,
        ],
        compiler_params=pltpu.CompilerParams(needs_layout_passes=False),
    )
    return fn(y, d0, d1, w0, w1)


def kernel(x, Wg, W1, W2, W3):
    x2 = x.reshape(NTOK, HD)
    wg_pad = jnp.zeros((128, HD), jnp.float32).at[:NE].set(Wg)
    w0, w1, e0, e1, r0, r1, off, bexp, aux = _run_router(x2, wg_pad)
    off16 = off[0, :16].astype(jnp.int32)
    bexp_i = bexp[:NB, 0].astype(jnp.int32)
    xpk = lax.bitcast_convert_type(
        x2.astype(jnp.bfloat16).reshape(NTOK, HD // 2, 2), jnp.int32)
    xs_pk, d0, d1 = _run_dispatch(xpk,
                                  e0.reshape(NTOK), e1.reshape(NTOK),
                                  r0.reshape(NTOK), r1.reshape(NTOK), off16)
    xs = lax.bitcast_convert_type(xs_pk, jnp.bfloat16).reshape(PAD, HD)
    y = _run_ffn(bexp_i, xs, W1.astype(jnp.bfloat16),
                 W3.astype(jnp.bfloat16), W2.astype(jnp.bfloat16))
    out2 = _run_combine(y, d0, d1, w0.reshape(NTOK), w1.reshape(NTOK))
    return out2.reshape(1, NTOK, HD), aux[0, 0]
